# baseline (speedup 1.0000x reference)
import jax
import jax.numpy as jnp
from jax import lax
from jax.experimental import pallas as pl
from jax.experimental.pallas import tpu as pltpu, tpu_sc as plsc

VOCAB = 100000
DIM = 1024
NPOS = 2048
BATCH = 4

NC = 2
NS = 16
NW = NC * NS
LANES = 16

POS_PER_W = NPOS // NW
CP = 8
NSTEP = POS_PER_W // CP
DVEC = DIM // LANES


def _take(x, idx):
    return lax.gather(
        x, idx[:, None],
        dimension_numbers=lax.GatherDimensionNumbers(
            offset_dims=(), collapsed_slice_dims=(0,), start_index_map=(0,)
        ),
        slice_sizes=(1,),
        mode=lax.GatherScatterMode.PROMISE_IN_BOUNDS,
    )


def _body(tok_hbm, w_hbm, pos_hbm, out_hbm,
          idx_v, row1_v, a_v, b_v, p0v, p1v,
          r00, r01, r02, r03, r10, r11, r12, r13,
          rsem, pg0, pg1,
          g00, g01, g02, g03, g10, g11, g12, g13,
          s00, s01, s02, s03, s10, s11, s12, s13):
    wid = lax.axis_index("s") * NC + lax.axis_index("c")
    p_base = wid * POS_PER_W
    rows = ((r00, r01, r02, r03), (r10, r11, r12, r13))
    posb = (p0v, p1v)
    psem = (pg0, pg1)
    gsem = ((g00, g01, g02, g03), (g10, g11, g12, g13))
    ssem = ((s00, s01, s02, s03), (s10, s11, s12, s13))

    idx_copies = [
        pltpu.async_copy(
            tok_hbm.at[pl.ds(b * NPOS + p_base, POS_PER_W)],
            idx_v.at[pl.ds(b * POS_PER_W, POS_PER_W)],
            ssem[1][b],
        )
        for b in range(BATCH)
    ]
    row1_copy = pltpu.async_copy(pos_hbm.at[pl.ds(DIM, DIM)], row1_v, rsem)

    def load_seed(c):
        return pltpu.async_copy(
            pos_hbm.at[pl.ds((p_base + c * CP) * DIM, DIM)],
            posb[c % 2], psem[c % 2],
        )

    def gather(c, b):
        q = c % 2
        return pltpu.async_copy(
            w_hbm.at[idx_v.at[pl.ds(b * POS_PER_W + c * CP, CP)]],
            rows[q][b], gsem[q][b],
        )

    pos_pending = load_seed(0)
    gathers = [[None] * BATCH, [None] * BATCH]
    stores = [[None] * BATCH, [None] * BATCH]
    for b in range(BATCH):
        idx_copies[b].wait()
        gathers[0][b] = gather(0, b)

    iota = lax.iota(jnp.int32, LANES)
    swap_idx = iota ^ 1
    dup_odd = iota | 1
    dup_even = iota & 14
    sign = jnp.where((iota & 1) == 1, -1.0, 1.0).astype(jnp.float32)
    row1_copy.wait()
    for d in range(DVEC):
        r1 = row1_v[pl.ds(d * LANES, LANES)]
        a_v[pl.ds(d * LANES, LANES)] = _take(r1, dup_odd)
        b_v[pl.ds(d * LANES, LANES)] = _take(r1, dup_even) * sign

    for c in range(NSTEP):
        q = c % 2
        nq = 1 - q
        pos_pending.wait()
        if c + 1 < NSTEP:
            pos_pending = load_seed(c + 1)
            for b in range(BATCH):
                if stores[nq][b] is not None:
                    stores[nq][b].wait()
                    stores[nq][b] = None
                gathers[nq][b] = gather(c + 1, b)
        for b in range(BATCH):
            gathers[q][b].wait()

        sv = posb[q]
        rbufs = rows[q]

        @pl.loop(0, DVEC)
        def _dloop(d):
            off = d * LANES
            a = a_v[pl.ds(off, LANES)]
            bb = b_v[pl.ds(off, LANES)]
            v0 = sv[pl.ds(off, LANES)]

            @pl.loop(0, CP, init_carry=v0)
            def _row(r, v):
                for b in range(BATCH):
                    plsc.addupdate(rbufs[b].at[r, pl.ds(off, LANES)], v)
                return v * a + _take(v, swap_idx) * bb

        for b in range(BATCH):
            stores[q][b] = pltpu.async_copy(
                rbufs[b],
                out_hbm.at[pl.ds(b * NPOS + p_base + c * CP, CP)],
                ssem[q][b],
            )
    for q in range(2):
        for b in range(BATCH):
            if stores[q][b] is not None:
                stores[q][b].wait()


@jax.jit
def _embed(tokens, W, pos_flat):
    mesh = plsc.VectorSubcoreMesh(
        core_axis_name="c", subcore_axis_name="s",
        num_cores=NC, num_subcores=NS,
    )
    run = pl.kernel(
        _body,
        out_type=jax.ShapeDtypeStruct((BATCH * NPOS, DIM), jnp.float32),
        mesh=mesh,
        scratch_types=[
            pltpu.VMEM((BATCH * POS_PER_W,), jnp.int32),
            pltpu.VMEM((DIM,), jnp.float32),
            pltpu.VMEM((DIM,), jnp.float32),
            pltpu.VMEM((DIM,), jnp.float32),
            pltpu.VMEM((DIM,), jnp.float32),
            pltpu.VMEM((DIM,), jnp.float32),
        ] + [pltpu.VMEM((CP, DIM), jnp.float32)] * 8
          + [pltpu.SemaphoreType.DMA] * 19,
    )
    out = run(tokens.reshape(-1), W, pos_flat)
    return out.reshape(BATCH, NPOS, DIM)


def kernel(tokens, W, pos_enc):
    return _embed(tokens.astype(jnp.int32), W, pos_enc.reshape(-1))


# speedup vs baseline: 1.0428x; 1.0428x over previous
import jax
import jax.numpy as jnp
from jax import lax
from jax.experimental import pallas as pl
from jax.experimental.pallas import tpu as pltpu, tpu_sc as plsc

VOCAB = 100000
DIM = 1024
NPOS = 2048
BATCH = 4

NC = 2
NS = 16
NW = NC * NS
LANES = 16

POS_PER_W = NPOS // NW
CP = 8
NSTEP = POS_PER_W // CP
DVEC = DIM // LANES


def _take(x, idx):
    return lax.gather(
        x, idx[:, None],
        dimension_numbers=lax.GatherDimensionNumbers(
            offset_dims=(), collapsed_slice_dims=(0,), start_index_map=(0,)
        ),
        slice_sizes=(1,),
        mode=lax.GatherScatterMode.PROMISE_IN_BOUNDS,
    )


def _body(tok_hbm, w_hbm, pos_hbm, out_hbm,
          idx_v, row1_v, a_v, b_v, p0v, p1v,
          r00, r01, r02, r03, r10, r11, r12, r13,
          rsem, pg0, pg1,
          g00, g01, g02, g03, g10, g11, g12, g13,
          s00, s01, s02, s03, s10, s11, s12, s13):
    wid = lax.axis_index("s") * NC + lax.axis_index("c")
    p_base = wid * POS_PER_W
    rows = ((r00, r01, r02, r03), (r10, r11, r12, r13))
    posb = (p0v, p1v)
    psem = (pg0, pg1)
    gsem = ((g00, g01, g02, g03), (g10, g11, g12, g13))
    ssem = ((s00, s01, s02, s03), (s10, s11, s12, s13))

    idx_copies = [
        pltpu.async_copy(
            tok_hbm.at[pl.ds(b * NPOS + p_base, POS_PER_W)],
            idx_v.at[pl.ds(b * POS_PER_W, POS_PER_W)],
            ssem[1][b],
        )
        for b in range(BATCH)
    ]
    row1_copy = pltpu.async_copy(pos_hbm.at[pl.ds(DIM, DIM)], row1_v, rsem)

    def load_seed(c):
        return pltpu.async_copy(
            pos_hbm.at[pl.ds((p_base + c * CP) * DIM, DIM)],
            posb[c % 2], psem[c % 2],
        )

    def gather(c, b):
        q = c % 2
        return pltpu.async_copy(
            w_hbm.at[idx_v.at[pl.ds(b * POS_PER_W + c * CP, CP)]],
            rows[q][b], gsem[q][b],
        )

    pos_pending = load_seed(0)
    gathers = [[None] * BATCH, [None] * BATCH]
    stores = [[None] * BATCH, [None] * BATCH]
    for b in range(BATCH):
        idx_copies[b].wait()
        gathers[0][b] = gather(0, b)

    iota = lax.iota(jnp.int32, LANES)
    swap_idx = iota ^ 1
    dup_odd = iota | 1
    dup_even = iota & 14
    sign = jnp.where((iota & 1) == 1, -1.0, 1.0).astype(jnp.float32)
    row1_copy.wait()
    for d in range(DVEC):
        r1 = row1_v[pl.ds(d * LANES, LANES)]
        a_v[pl.ds(d * LANES, LANES)] = _take(r1, dup_odd)
        b_v[pl.ds(d * LANES, LANES)] = _take(r1, dup_even) * sign

    for c in range(NSTEP):
        q = c % 2
        nq = 1 - q
        pos_pending.wait()
        if c + 1 < NSTEP:
            pos_pending = load_seed(c + 1)
            for b in range(BATCH):
                if stores[nq][b] is not None:
                    stores[nq][b].wait()
                    stores[nq][b] = None
                gathers[nq][b] = gather(c + 1, b)
        for b in range(BATCH):
            gathers[q][b].wait()

        sv = posb[q]
        rbufs = rows[q]

        @pl.loop(0, DVEC // 8)
        def _dloop(blk):
            base = blk * (8 * LANES)
            a = [a_v[pl.ds(base + i * LANES, LANES)] for i in range(8)]
            bb = [b_v[pl.ds(base + i * LANES, LANES)] for i in range(8)]
            v0 = tuple(sv[pl.ds(base + i * LANES, LANES)] for i in range(8))

            @pl.loop(0, CP, init_carry=v0)
            def _row(r, vs):
                for i in range(8):
                    for b in range(BATCH):
                        plsc.addupdate(
                            rbufs[b].at[r, pl.ds(base + i * LANES, LANES)],
                            vs[i],
                        )
                return tuple(
                    vs[i] * a[i] + _take(vs[i], swap_idx) * bb[i]
                    for i in range(8)
                )

        for b in range(BATCH):
            stores[q][b] = pltpu.async_copy(
                rbufs[b],
                out_hbm.at[pl.ds(b * NPOS + p_base + c * CP, CP)],
                ssem[q][b],
            )
    for q in range(2):
        for b in range(BATCH):
            if stores[q][b] is not None:
                stores[q][b].wait()


@jax.jit
def _embed(tokens, W, pos_flat):
    mesh = plsc.VectorSubcoreMesh(
        core_axis_name="c", subcore_axis_name="s",
        num_cores=NC, num_subcores=NS,
    )
    run = pl.kernel(
        _body,
        out_type=jax.ShapeDtypeStruct((BATCH * NPOS, DIM), jnp.float32),
        mesh=mesh,
        scratch_types=[
            pltpu.VMEM((BATCH * POS_PER_W,), jnp.int32),
            pltpu.VMEM((DIM,), jnp.float32),
            pltpu.VMEM((DIM,), jnp.float32),
            pltpu.VMEM((DIM,), jnp.float32),
            pltpu.VMEM((DIM,), jnp.float32),
            pltpu.VMEM((DIM,), jnp.float32),
        ] + [pltpu.VMEM((CP, DIM), jnp.float32)] * 8
          + [pltpu.SemaphoreType.DMA] * 19,
    )
    out = run(tokens.reshape(-1), W, pos_flat)
    return out.reshape(BATCH, NPOS, DIM)


def kernel(tokens, W, pos_enc):
    return _embed(tokens.astype(jnp.int32), W, pos_enc.reshape(-1))


# speedup vs baseline: 1.0768x; 1.0327x over previous
import jax
import jax.numpy as jnp
from jax import lax
from jax.experimental import pallas as pl
from jax.experimental.pallas import tpu as pltpu, tpu_sc as plsc

VOCAB = 100000
DIM = 1024
NPOS = 2048
BATCH = 4

NC = 2
NS = 16
NW = NC * NS
LANES = 16

POS_PER_W = NPOS // NW
CP = 8
NSTEP = POS_PER_W // CP
DVEC = DIM // LANES


def _body(tok_hbm, w_hbm, pos_hbm, out_hbm,
          idx_v, p0v, p1v,
          r00, r01, r02, r03, r10, r11, r12, r13,
          pg0, pg1,
          g00, g01, g02, g03, g10, g11, g12, g13,
          s00, s01, s02, s03, s10, s11, s12, s13):
    wid = lax.axis_index("s") * NC + lax.axis_index("c")
    p_base = wid * POS_PER_W
    rows = ((r00, r01, r02, r03), (r10, r11, r12, r13))
    posb = (p0v, p1v)
    psem = (pg0, pg1)
    gsem = ((g00, g01, g02, g03), (g10, g11, g12, g13))
    ssem = ((s00, s01, s02, s03), (s10, s11, s12, s13))

    idx_copies = [
        pltpu.async_copy(
            tok_hbm.at[pl.ds(b * NPOS + p_base, POS_PER_W)],
            idx_v.at[pl.ds(b * POS_PER_W, POS_PER_W)],
            ssem[1][b],
        )
        for b in range(BATCH)
    ]

    def load_pos(c):
        return pltpu.async_copy(
            pos_hbm.at[pl.ds(p_base + c * CP, CP)], posb[c % 2], psem[c % 2]
        )

    def gather(c, b):
        q = c % 2
        return pltpu.async_copy(
            w_hbm.at[idx_v.at[pl.ds(b * POS_PER_W + c * CP, CP)]],
            rows[q][b], gsem[q][b],
        )

    pos_pending = load_pos(0)
    gathers = [[None] * BATCH, [None] * BATCH]
    stores = [[None] * BATCH, [None] * BATCH]
    for b in range(BATCH):
        idx_copies[b].wait()
        gathers[0][b] = gather(0, b)

    for c in range(NSTEP):
        q = c % 2
        nq = 1 - q
        pos_pending.wait()
        if c + 1 < NSTEP:
            pos_pending = load_pos(c + 1)
            for b in range(BATCH):
                if stores[nq][b] is not None:
                    stores[nq][b].wait()
                    stores[nq][b] = None
                gathers[nq][b] = gather(c + 1, b)
        for b in range(BATCH):
            gathers[q][b].wait()

        pv = posb[q]
        rbufs = rows[q]

        @pl.loop(0, CP)
        def _row(r):
            for d in range(DVEC):
                off = d * LANES
                pvec = pv[r, pl.ds(off, LANES)]
                for b in range(BATCH):
                    plsc.addupdate(rbufs[b].at[r, pl.ds(off, LANES)], pvec)

        for b in range(BATCH):
            stores[q][b] = pltpu.async_copy(
                rbufs[b],
                out_hbm.at[pl.ds(b * NPOS + p_base + c * CP, CP)],
                ssem[q][b],
            )
    for q in range(2):
        for b in range(BATCH):
            if stores[q][b] is not None:
                stores[q][b].wait()


@jax.jit
def _embed(tokens, W, pos_enc):
    mesh = plsc.VectorSubcoreMesh(
        core_axis_name="c", subcore_axis_name="s",
        num_cores=NC, num_subcores=NS,
    )
    run = pl.kernel(
        _body,
        out_type=jax.ShapeDtypeStruct((BATCH * NPOS, DIM), jnp.float32),
        mesh=mesh,
        scratch_types=[
            pltpu.VMEM((BATCH * POS_PER_W,), jnp.int32),
            pltpu.VMEM((CP, DIM), jnp.float32),
            pltpu.VMEM((CP, DIM), jnp.float32),
        ] + [pltpu.VMEM((CP, DIM), jnp.float32)] * 8
          + [pltpu.SemaphoreType.DMA] * 18,
    )
    out = run(tokens.reshape(-1), W, pos_enc)
    return out.reshape(BATCH, NPOS, DIM)


def kernel(tokens, W, pos_enc):
    return _embed(tokens.astype(jnp.int32), W, pos_enc)
